# Initial kernel scaffold; baseline (speedup 1.0000x reference)
import jax
import jax.numpy as jnp
from jax.experimental import pallas as pl


def kernel(edge_index, senders_pos, receivers_pos, edge_dx_, edge_dt_, edge_attr, vector_a, vector_b, vector_c, senders_v_t_, senders_v_tm1_, senders_w_t_, receivers_v_t_, receivers_v_tm1_, receivers_w_t_, node_latent, params):
    raise NotImplementedError("write your pallas kernel here")


# speedup vs baseline: 1.2368x; 1.2368x over previous
import functools

import jax
import jax.numpy as jnp
from jax import lax
from jax.experimental import pallas as pl
from jax.experimental.pallas import tpu as pltpu
from jax.experimental.pallas import tpu_sc as plsc

F32 = jnp.float32

L = 128
NODE_BLK = 2000
EDGE_BLK = 512
GC = 80
SC_WORKERS = 32
FIN_BLK = 2000


def _mm(a, w):
    return lax.dot_general(a, w, (((1,), (0,)), ((), ())),
                           preferred_element_type=F32)


def _ln(o, g, be):
    m = jnp.mean(o, axis=1, keepdims=True)
    c = o - m
    v = jnp.mean(c * c, axis=1, keepdims=True)
    return c * lax.rsqrt(v + 1e-5) * g + be


def _relu(x):
    return jnp.maximum(x, 0.0)


def _div3(x):
    return (x * 21846) >> 16


def _node_body(nl_ref, w1_ref, b1_ref, w2_ref, b2_ref, nscal_ref):
    nl = nl_ref[...]
    h = _relu(_mm(nl, w1_ref[...]) + b1_ref[...])
    nscal_ref[...] = _mm(h, w2_ref[...]) + b2_ref[...]


def _node_stage(node_latent, w1cat, b1cat, w2cat, b2cat):
    n = node_latent.shape[0]
    grid = n // NODE_BLK
    full = lambda shape: pl.BlockSpec(shape, lambda i: tuple(0 for _ in shape))
    return pl.pallas_call(
        _node_body,
        grid=(grid,),
        in_specs=[
            pl.BlockSpec((NODE_BLK, L), lambda i: (i, 0)),
            full((L, 4 * L)), full((1, 4 * L)), full((4 * L, 4)), full((1, 4)),
        ],
        out_specs=pl.BlockSpec((NODE_BLK, 4), lambda i: (i, 0)),
        out_shape=jax.ShapeDtypeStruct((n, 4), F32),
    )(node_latent, w1cat, b1cat, w2cat, b2cat)


def _gather_stage(tbl, sidx, ridx, nw_flat, sp_flat, rp_flat):
    e = sidx.shape[0]
    n = nw_flat.shape[0]
    epw = e // SC_WORKERS
    mesh = plsc.VectorSubcoreMesh(core_axis_name="c", subcore_axis_name="s")

    @functools.partial(
        pl.kernel, mesh=mesh,
        compiler_params=pltpu.CompilerParams(needs_layout_passes=False),
        out_type=[
            jax.ShapeDtypeStruct((e, L), F32),
            jax.ShapeDtypeStruct((e, L), F32),
            jax.ShapeDtypeStruct((3 * e,), F32),
        ],
        scratch_types=[
            pltpu.VMEM((n,), F32),
            pltpu.VMEM((GC,), jnp.int32),
            pltpu.VMEM((GC,), jnp.int32),
            pltpu.VMEM((GC, L), F32),
            pltpu.VMEM((GC, L), F32),
            pltpu.VMEM((GC,), F32),
            pltpu.VMEM((GC,), F32),
            pltpu.VMEM((3 * GC,), F32),
            pltpu.VMEM((3 * GC,), F32),
            pltpu.VMEM((3 * GC,), F32),
            pltpu.SemaphoreType.DMA,
            pltpu.SemaphoreType.DMA,
        ],
    )
    def k(tbl_hbm, s_hbm, r_hbm, nw_hbm, sp_hbm, rp_hbm,
          gs_hbm, gr_hbm, r0_hbm,
          nw_v, sv, rv, srows, rrows, wsv, wrv, spv, rpv, r0v, sem1, sem2):
        iota = lax.iota(jnp.int32, 16)
        wid = lax.axis_index("s") * 2 + lax.axis_index("c")
        base = wid * epw
        pltpu.sync_copy(nw_hbm, nw_v)

        def body(i, carry):
            b = base + i * GC
            pltpu.sync_copy(s_hbm.at[pl.ds(b, GC)], sv)
            pltpu.sync_copy(r_hbm.at[pl.ds(b, GC)], rv)
            cp1 = pltpu.async_copy(tbl_hbm.at[sv], srows, sem1)
            cp2 = pltpu.async_copy(tbl_hbm.at[rv], rrows, sem2)
            pltpu.sync_copy(sp_hbm.at[pl.ds(b * 3, 3 * GC)], spv)
            pltpu.sync_copy(rp_hbm.at[pl.ds(b * 3, 3 * GC)], rpv)
            for sub in range(GC // 16):
                o = sub * 16
                s16 = sv[pl.ds(o, 16)]
                r16 = rv[pl.ds(o, 16)]
                nws = plsc.load_gather(nw_v, [s16])
                nwr = plsc.load_gather(nw_v, [r16])
                inv = 1.0 / (nws + nwr)
                wsv[pl.ds(o, 16)] = nws * inv
                wrv[pl.ds(o, 16)] = nwr * inv
            for j in range(3 * GC // 16):
                o = j * 16
                e16 = _div3(iota + o)
                ws = plsc.load_gather(wsv, [e16])
                wr = plsc.load_gather(wrv, [e16])
                r0v[pl.ds(o, 16)] = (ws * spv[pl.ds(o, 16)]
                                     + wr * rpv[pl.ds(o, 16)])
            pltpu.sync_copy(r0v, r0_hbm.at[pl.ds(b * 3, 3 * GC)])
            cp1.wait()
            cp2.wait()
            pltpu.sync_copy(srows, gs_hbm.at[pl.ds(b, GC)])
            pltpu.sync_copy(rrows, gr_hbm.at[pl.ds(b, GC)])
            return carry

        lax.fori_loop(0, epw // GC, body, 0)

    return k(tbl, sidx, ridx, nw_flat, sp_flat, rp_flat)


def _edge_body(gs_ref, gr_ref, r0_ref, rp_ref, dx_ref, dt_ref, ea_ref,
               va_ref, vb_ref, vc_ref, svt_ref, svtm_ref, swt_ref,
               rvt_ref, rvtm_ref, rwt_ref,
               efW1_ref, efb1_ref, efW2_ref, efb2_ref, efg_ref, efbe_ref,
               eW1_ref, eb1_ref, eW2_ref, eb2_ref, eg_ref, ebe_ref,
               iW1a_ref, iW1b_ref, iW1c_ref, ib1_ref, iW2_ref, ib2_ref,
               ig_ref, ibe_ref, catW1_ref, catb1_ref, combW2_ref, combb2_ref,
               il_ref, ft_ref):
    B = gs_ref.shape[0]

    def dot3(u, v):
        return (u[:, 0:1] * v[:, 0:1] + u[:, 1:2] * v[:, 1:2]
                + u[:, 2:3] * v[:, 2:3])

    def b16(x):
        return x.astype(jnp.bfloat16).astype(F32)

    va, vb, vc = va_ref[...], vb_ref[...], vc_ref[...]
    vecs = (va, vb, vc)
    efW1 = b16(efW1_ref[...])

    acc_s = jnp.zeros((B, L), F32)
    acc_r = acc_s
    k = 0
    for u_ref in (svt_ref, svtm_ref, swt_ref):
        u = u_ref[...]
        for v in vecs:
            acc_s = acc_s + b16(dot3(u, v)) * efW1[k:k + 1, :]
            k += 1
    k = 0
    for u_ref in (rvt_ref, rvtm_ref, rwt_ref):
        u = u_ref[...]
        for v in vecs:
            acc_r = acc_r - b16(dot3(u, v)) * efW1[k:k + 1, :]
            k += 1
    efb1 = efb1_ref[...]
    acc_s = acc_s + efb1
    acc_r = acc_r + efb1

    efW2, efb2, efg, efbe = efW2_ref[...], efb2_ref[...], efg_ref[...], efbe_ref[...]
    s_lat = _ln(_mm(_relu(acc_s), efW2) + efb2, efg, efbe)
    r_lat = _ln(_mm(_relu(acc_r), efW2) + efb2, efg, efbe)

    dx, dt = dx_ref[...], dt_ref[...]
    dxn = jnp.sqrt(dot3(dx, dx))
    dtn = jnp.sqrt(dot3(dt, dt))
    eW1 = b16(eW1_ref[...])
    acc_e = (b16(dxn) * eW1[0:1, :] + b16(dtn) * eW1[1:2, :]
             + b16(ea_ref[...][:, 0:1]) * eW1[2:3, :] + eb1_ref[...])
    e_lat = _ln(_mm(_relu(acc_e), eW2_ref[...]) + eb2_ref[...],
                eg_ref[...], ebe_ref[...])

    pre = (_mm(s_lat + r_lat, iW1a_ref[...])
           + _mm(gs_ref[...] + gr_ref[...], iW1b_ref[...])
           + _mm(e_lat, iW1c_ref[...]) + ib1_ref[...])
    il = _ln(_mm(_relu(pre), iW2_ref[...]) + ib2_ref[...],
             ig_ref[...], ibe_ref[...])
    il_ref[...] = il

    hcat = _relu(_mm(il, catW1_ref[...]) + catb1_ref[...])
    cf = _mm(hcat, combW2_ref[...]) + combb2_ref[...]

    f = [cf[:, 0:1] * va[:, c:c + 1] + cf[:, 1:2] * vb[:, c:c + 1]
         + cf[:, 2:3] * vc[:, c:c + 1] for c in range(3)]
    a = [cf[:, 3:4] * va[:, c:c + 1] + cf[:, 4:5] * vb[:, c:c + 1]
         + cf[:, 5:6] * vc[:, c:c + 1] for c in range(3)]
    lam = cf[:, 6:7]

    rp, r0 = rp_ref[...], r0_ref[...]
    d = [rp[:, c:c + 1] - r0[:, c:c + 1] for c in range(3)]
    cr = [d[1] * f[2] - d[2] * f[1],
          d[2] * f[0] - d[0] * f[2],
          d[0] * f[1] - d[1] * f[0]]

    zero = jnp.zeros((B, 1), F32)
    ft_ref[0, :, 0:1] = f[0]
    ft_ref[0, :, 1:2] = f[1]
    ft_ref[0, :, 2:3] = f[2]
    ft_ref[0, :, 3:4] = zero
    ft_ref[1, :, 0:1] = a[0] - cr[0] * lam
    ft_ref[1, :, 1:2] = a[1] - cr[1] * lam
    ft_ref[1, :, 2:3] = a[2] - cr[2] * lam
    ft_ref[1, :, 3:4] = zero


def _edge_stage(gs, gr, r0, edge_in, weights):
    e = gs.shape[0]
    grid = e // EDGE_BLK
    full = lambda shape: pl.BlockSpec(shape, lambda i: tuple(0 for _ in shape))
    eb = lambda w: pl.BlockSpec((EDGE_BLK, w), lambda i: (i, 0))
    in_specs = [eb(L), eb(L), eb(3)]
    in_specs += [eb(3)] * 3 + [eb(1)] + [eb(3)] * 9
    in_specs += [full(w.shape) for w in weights]
    return pl.pallas_call(
        _edge_body,
        grid=(grid,),
        in_specs=in_specs,
        out_specs=[
            pl.BlockSpec((EDGE_BLK, L), lambda i: (i, 0)),
            pl.BlockSpec((2, EDGE_BLK, 4), lambda i: (0, i, 0)),
        ],
        out_shape=[
            jax.ShapeDtypeStruct((e, L), F32),
            jax.ShapeDtypeStruct((2, e, 4), F32),
        ],
    )(gs, gr, r0, *edge_in, *weights)


def _scatter_stage(ridx, ft4, n):
    e = ridx.shape[0]
    ept = e // 16
    slc = -(-(3 * n) // 16)
    slc = -(-slc // 128) * 128
    acc_len = 16 * slc
    mesh = plsc.VectorSubcoreMesh(core_axis_name="c", subcore_axis_name="s")

    @functools.partial(
        pl.kernel, mesh=mesh,
        compiler_params=pltpu.CompilerParams(needs_layout_passes=False),
        out_type=jax.ShapeDtypeStruct((2 * acc_len,), F32),
        scratch_types=[
            pltpu.VMEM((GC,), jnp.int32),
            pltpu.VMEM((4 * GC,), F32),
            pltpu.VMEM((acc_len,), F32),
            pltpu.VMEM((slc,), F32),
            pltpu.VMEM_SHARED((16 * acc_len,), F32),
        ],
    )
    def k(ridx_hbm, ft_hbm, out_hbm, idxv, valv, acc, tmp, shared):
        iota = lax.iota(jnp.int32, 16)
        c = lax.axis_index("c")
        t = lax.axis_index("s")
        z = jnp.zeros((16,), F32)

        def zb(i, carry):
            acc[pl.ds(i * 16, 16)] = z
            return carry

        lax.fori_loop(0, acc_len // 16, zb, 0)
        base = t * ept

        def body(i, carry):
            b = base + i * GC
            pltpu.sync_copy(ridx_hbm.at[pl.ds(b, GC)], idxv)
            pltpu.sync_copy(ft_hbm.at[pl.ds(c * (4 * e) + b * 4, 4 * GC)],
                            valv)
            for sub in range(GC // 16):
                r16 = idxv[pl.ds(sub * 16, 16)]
                a16 = r16 * 3
                for comp in range(3):
                    v16 = plsc.load_gather(valv, [iota * 4 + (sub * 64 + comp)])
                    plsc.addupdate_scatter(acc, [a16 + comp], v16)
            return carry

        lax.fori_loop(0, ept // GC, body, 0)
        pltpu.sync_copy(acc, shared.at[pl.ds(t * acc_len, acc_len)])
        plsc.subcore_barrier()
        s0 = t * slc
        pltpu.sync_copy(shared.at[pl.ds(s0, slc)], tmp)

        def red(j, carry):
            pltpu.sync_copy(shared.at[pl.ds(j * acc_len + s0, slc)],
                            acc.at[pl.ds(0, slc)])

            def addl(q, carry2):
                tmp[pl.ds(q * 16, 16)] = (tmp[pl.ds(q * 16, 16)]
                                          + acc[pl.ds(q * 16, 16)])
                return carry2

            lax.fori_loop(0, slc // 16, addl, 0)
            return carry

        lax.fori_loop(1, 16, red, 0)
        pltpu.sync_copy(tmp, out_hbm.at[pl.ds(c * acc_len + s0, slc)])

    return k(ridx, ft4), acc_len


def _fin_body(accf_ref, acct_ref, ns_ref, dv_ref, dw_ref):
    minv = ns_ref[:, 1:2]
    iinv = ns_ref[:, 2:3]
    dvx = ns_ref[:, 3:4]
    dv_ref[...] = minv * accf_ref[...] + dvx
    dw_ref[...] = iinv * acct_ref[...]


def _fin_stage(accf, acct, nscal):
    n = nscal.shape[0]
    grid = n // FIN_BLK
    return pl.pallas_call(
        _fin_body,
        grid=(grid,),
        in_specs=[
            pl.BlockSpec((FIN_BLK, 3), lambda i: (i, 0)),
            pl.BlockSpec((FIN_BLK, 3), lambda i: (i, 0)),
            pl.BlockSpec((FIN_BLK, 4), lambda i: (i, 0)),
        ],
        out_specs=[
            pl.BlockSpec((FIN_BLK, 3), lambda i: (i, 0)),
            pl.BlockSpec((FIN_BLK, 3), lambda i: (i, 0)),
        ],
        out_shape=[
            jax.ShapeDtypeStruct((n, 3), F32),
            jax.ShapeDtypeStruct((n, 3), F32),
        ],
    )(accf, acct, nscal)


def _prep_node_weights(p):
    order = ("nw", "minv", "iinv", "dvext")
    w1cat = jnp.concatenate([p[k]["W1"] for k in order], axis=1)
    b1cat = jnp.concatenate([p[k]["b1"] for k in order])[None, :]
    w2cat = jnp.zeros((4 * L, 4), F32)
    for i, k in enumerate(order):
        w2cat = w2cat.at[i * L:(i + 1) * L, i:i + 1].set(p[k]["W2"])
    b2cat = jnp.concatenate([p[k]["b2"] for k in order])[None, :]
    return w1cat, b1cat, w2cat, b2cat


def _prep_edge_weights(p):
    ef, ed, it = p["edge_feat"], p["edge"], p["inter"]
    catW1 = jnp.concatenate([p["i1"]["W1"], p["i2"]["W1"], p["fsc"]["W1"]],
                            axis=1)
    catb1 = jnp.concatenate([p["i1"]["b1"], p["i2"]["b1"],
                             p["fsc"]["b1"]])[None, :]
    combW2 = jnp.zeros((3 * L, 8), F32)
    combW2 = combW2.at[0:L, 0:3].set(p["i1"]["W2"])
    combW2 = combW2.at[L:2 * L, 3:6].set(p["i2"]["W2"])
    combW2 = combW2.at[2 * L:3 * L, 6:7].set(p["fsc"]["W2"])
    combb2 = jnp.concatenate([p["i1"]["b2"], p["i2"]["b2"], p["fsc"]["b2"],
                              jnp.zeros((1,), F32)])[None, :]
    r1 = lambda x: x[None, :]
    return (ef["W1"], r1(ef["b1"]), ef["W2"], r1(ef["b2"]), r1(ef["g"]),
            r1(ef["be"]),
            ed["W1"], r1(ed["b1"]), ed["W2"], r1(ed["b2"]), r1(ed["g"]),
            r1(ed["be"]),
            it["W1"][0:L], it["W1"][L:2 * L], it["W1"][2 * L:3 * L],
            r1(it["b1"]), it["W2"],
            r1(it["b2"]), r1(it["g"]), r1(it["be"]),
            catW1, catb1, combW2, combb2)


def kernel(edge_index, senders_pos, receivers_pos, edge_dx_, edge_dt_,
           edge_attr, vector_a, vector_b, vector_c, senders_v_t_,
           senders_v_tm1_, senders_w_t_, receivers_v_t_, receivers_v_tm1_,
           receivers_w_t_, node_latent, params):
    n = node_latent.shape[0]
    ei = edge_index.astype(jnp.int32)
    senders, receivers = ei[0], ei[1]

    nw1, nb1, nw2, nb2 = _prep_node_weights(params)
    nl32 = node_latent.astype(F32)
    nscal = _node_stage(nl32, nw1, nb1, nw2, nb2)

    gs, gr, r0_flat = _gather_stage(
        nl32, senders, receivers, nscal[:, 0],
        senders_pos.reshape(-1), receivers_pos.reshape(-1))
    r0 = r0_flat.reshape(-1, 3)

    edge_in = (receivers_pos, edge_dx_, edge_dt_, edge_attr,
               vector_a, vector_b, vector_c, senders_v_t_, senders_v_tm1_,
               senders_w_t_, receivers_v_t_, receivers_v_tm1_,
               receivers_w_t_)
    ew = _prep_edge_weights(params)
    il, ft = _edge_stage(gs, gr, r0, edge_in, ew)

    acc2, acc_len = _scatter_stage(receivers, ft.reshape(-1), n)
    accf = acc2[:3 * n].reshape(n, 3)
    acct = acc2[acc_len:acc_len + 3 * n].reshape(n, 3)

    dv, dw = _fin_stage(accf, acct, nscal)
    return (dv, dw, il)


# speedup vs baseline: 4.2092x; 3.4032x over previous
import functools

import jax
import jax.numpy as jnp
from jax import lax
from jax.experimental import pallas as pl
from jax.experimental.pallas import tpu as pltpu
from jax.experimental.pallas import tpu_sc as plsc

F32 = jnp.float32

L = 128
NODE_BLK = 2000
EDGE_BLK = 640
GC = 80
SC_WORKERS = 32
FIN_BLK = 2000
NSTK = 37


def _mm(a, w):
    return lax.dot_general(a, w, (((1,), (0,)), ((), ())),
                           preferred_element_type=F32)


def _mm_t(d, w):
    return lax.dot_general(d, w, (((0,), (0,)), ((), ())),
                           preferred_element_type=F32)


def _mm_h(w, a):
    return lax.dot_general(w, a, (((0,), (1,)), ((), ())),
                           preferred_element_type=F32)


def _ln(o, g, be):
    m = jnp.mean(o, axis=1, keepdims=True)
    c = o - m
    v = jnp.mean(c * c, axis=1, keepdims=True)
    return c * lax.rsqrt(v + 1e-5) * g + be


def _relu(x):
    return jnp.maximum(x, 0.0)


def _node_body(nl_ref, w1_ref, b1_ref, w2_ref, b2_ref, nscal_ref):
    nl = nl_ref[...]
    h = _relu(_mm(nl, w1_ref[...]) + b1_ref[...])
    nscal_ref[...] = _mm(h, w2_ref[...]) + b2_ref[...]


def _node_stage(node_latent, w1cat, b1cat, w2cat, b2cat):
    n = node_latent.shape[0]
    grid = n // NODE_BLK
    full = lambda shape: pl.BlockSpec(shape, lambda i: tuple(0 for _ in shape))
    return pl.pallas_call(
        _node_body,
        grid=(grid,),
        in_specs=[
            pl.BlockSpec((NODE_BLK, L), lambda i: (i, 0)),
            full((L, 4 * L)), full((1, 4 * L)), full((4 * L, 4)), full((1, 4)),
        ],
        out_specs=pl.BlockSpec((NODE_BLK, 4), lambda i: (i, 0)),
        out_shape=jax.ShapeDtypeStruct((n, 4), F32),
    )(node_latent, w1cat, b1cat, w2cat, b2cat)


def _gather_stage(tbl, sidx, ridx, nw_flat, spT, rpT):
    e = sidx.shape[0]
    n = nw_flat.shape[0]
    epw = e // SC_WORKERS
    mesh = plsc.VectorSubcoreMesh(core_axis_name="c", subcore_axis_name="s")

    @functools.partial(
        pl.kernel, mesh=mesh,
        compiler_params=pltpu.CompilerParams(needs_layout_passes=False),
        out_type=[
            jax.ShapeDtypeStruct((e, L), F32),
            jax.ShapeDtypeStruct((e, L), F32),
            jax.ShapeDtypeStruct((3 * e,), F32),
        ],
        scratch_types=[
            pltpu.VMEM((n,), F32),
            pltpu.VMEM((GC,), jnp.int32),
            pltpu.VMEM((GC,), jnp.int32),
            pltpu.VMEM((GC, L), F32),
            pltpu.VMEM((GC, L), F32),
            pltpu.VMEM((GC,), F32),
            pltpu.VMEM((GC,), F32),
            pltpu.VMEM((GC,), F32),
            pltpu.VMEM((GC,), F32),
            pltpu.VMEM((GC,), F32),
            pltpu.SemaphoreType.DMA,
            pltpu.SemaphoreType.DMA,
        ],
    )
    def k(tbl_hbm, s_hbm, r_hbm, nw_hbm, sp_hbm, rp_hbm,
          gs_hbm, gr_hbm, r0_hbm,
          nw_v, sv, rv, srows, rrows, wsv, wrv, spv, rpv, r0v, sem1, sem2):
        wid = lax.axis_index("s") * 2 + lax.axis_index("c")
        base = wid * epw
        pltpu.sync_copy(nw_hbm, nw_v)

        def body(i, carry):
            b = base + i * GC
            pltpu.sync_copy(s_hbm.at[pl.ds(b, GC)], sv)
            pltpu.sync_copy(r_hbm.at[pl.ds(b, GC)], rv)
            cp1 = pltpu.async_copy(tbl_hbm.at[sv], srows, sem1)
            cp2 = pltpu.async_copy(tbl_hbm.at[rv], rrows, sem2)
            for sub in range(GC // 16):
                o = sub * 16
                s16 = sv[pl.ds(o, 16)]
                r16 = rv[pl.ds(o, 16)]
                nws = plsc.load_gather(nw_v, [s16])
                nwr = plsc.load_gather(nw_v, [r16])
                inv = 1.0 / (nws + nwr)
                wsv[pl.ds(o, 16)] = nws * inv
                wrv[pl.ds(o, 16)] = nwr * inv
            for comp in range(3):
                pltpu.sync_copy(sp_hbm.at[pl.ds(comp * e + b, GC)], spv)
                pltpu.sync_copy(rp_hbm.at[pl.ds(comp * e + b, GC)], rpv)
                for sub in range(GC // 16):
                    o = sub * 16
                    r0v[pl.ds(o, 16)] = (wsv[pl.ds(o, 16)] * spv[pl.ds(o, 16)]
                                         + wrv[pl.ds(o, 16)] * rpv[pl.ds(o, 16)])
                pltpu.sync_copy(r0v, r0_hbm.at[pl.ds(comp * e + b, GC)])
            cp1.wait()
            cp2.wait()
            pltpu.sync_copy(srows, gs_hbm.at[pl.ds(b, GC)])
            pltpu.sync_copy(rrows, gr_hbm.at[pl.ds(b, GC)])
            return carry

        lax.fori_loop(0, epw // GC, body, 0)

    return k(tbl, sidx, ridx, nw_flat, spT, rpT)


def _edge_body(stk_ref, r0_ref, gs_ref, gr_ref,
               efW1_ref, efb1_ref, efW2_ref, efb2_ref, efg_ref, efbe_ref,
               eW1_ref, eb1_ref, eW2_ref, eb2_ref, eg_ref, ebe_ref,
               iW1a_ref, iW1b_ref, iW1c_ref, ib1_ref, iW2_ref, ib2_ref,
               ig_ref, ibe_ref, catW1_ref, catb1_ref, combW2_ref, combb2_ref,
               il_ref, ft_ref):
    B = gs_ref.shape[0]

    def row(k):
        return stk_ref[k, :]

    def rowm(ref, k):
        return ref[k, :].reshape(1, B)

    zrow = jnp.zeros((1, B), F32)

    def dstack(rows9):
        rows = [d.reshape(1, B) for d in rows9]
        return jnp.concatenate(rows + [zrow] * (16 - len(rows)), axis=0)

    uvecs = [[row(9 * g + 3 * i + c) for c in range(3)] for g in range(2)
             for i in range(3)]
    vvecs = [[row(18 + 3 * j + c) for c in range(3)] for j in range(3)]

    def dot3(u, v):
        return u[0] * v[0] + u[1] * v[1] + u[2] * v[2]

    ds = dstack([dot3(uvecs[i], vvecs[j])
                 for i in range(3) for j in range(3)])
    dr = dstack([-dot3(uvecs[3 + i], vvecs[j])
                 for i in range(3) for j in range(3)])

    acc_s = _mm_t(ds, efW1_ref[...]) + efb1_ref[...]
    acc_r = _mm_t(dr, efW1_ref[...]) + efb1_ref[...]

    efW2, efb2, efg, efbe = efW2_ref[...], efb2_ref[...], efg_ref[...], efbe_ref[...]
    s_lat = _ln(_mm(_relu(acc_s), efW2) + efb2, efg, efbe)
    r_lat = _ln(_mm(_relu(acc_r), efW2) + efb2, efg, efbe)

    dxc = [row(27 + c) for c in range(3)]
    dtc = [row(30 + c) for c in range(3)]
    dxn = jnp.sqrt(dot3(dxc, dxc))
    dtn = jnp.sqrt(dot3(dtc, dtc))
    de = jnp.concatenate([dxn.reshape(1, B), dtn.reshape(1, B),
                          rowm(stk_ref, 33)] + [zrow] * 5, axis=0)
    acc_e = _mm_t(de, eW1_ref[...]) + eb1_ref[...]
    e_lat = _ln(_mm(_relu(acc_e), eW2_ref[...]) + eb2_ref[...],
                eg_ref[...], ebe_ref[...])

    pre = (_mm(s_lat + r_lat, iW1a_ref[...])
           + _mm(gs_ref[...] + gr_ref[...], iW1b_ref[...])
           + _mm(e_lat, iW1c_ref[...]) + ib1_ref[...])
    il = _ln(_mm(_relu(pre), iW2_ref[...]) + ib2_ref[...],
             ig_ref[...], ibe_ref[...])
    il_ref[...] = il

    hcat = _relu(_mm(il, catW1_ref[...]) + catb1_ref[...])
    cfT = _mm_h(combW2_ref[...], hcat) + combb2_ref[...]

    cf = [cfT[j, :] for j in range(7)]
    va = [row(18 + c) for c in range(3)]
    vb = [row(21 + c) for c in range(3)]
    vc = [row(24 + c) for c in range(3)]
    f = [cf[0] * va[c] + cf[1] * vb[c] + cf[2] * vc[c] for c in range(3)]
    a = [cf[3] * va[c] + cf[4] * vb[c] + cf[5] * vc[c] for c in range(3)]
    lam = cf[6]

    rp = [row(34 + c) for c in range(3)]
    d = [rp[c] - r0_ref[c, :] for c in range(3)]
    cr = [d[1] * f[2] - d[2] * f[1],
          d[2] * f[0] - d[0] * f[2],
          d[0] * f[1] - d[1] * f[0]]

    for c in range(3):
        ft_ref[c, :] = f[c]
        ft_ref[3 + c, :] = a[c] - cr[c] * lam


def _edge_stage(stk, r0T, gs, gr, weights):
    e = gs.shape[0]
    B = EDGE_BLK
    grid = e // B
    full = lambda shape: pl.BlockSpec(shape, lambda i: tuple(0 for _ in shape))
    in_specs = [
        pl.BlockSpec((NSTK, B), lambda i: (0, i)),
        pl.BlockSpec((3, B), lambda i: (0, i)),
        pl.BlockSpec((B, L), lambda i: (i, 0)),
        pl.BlockSpec((B, L), lambda i: (i, 0)),
    ]
    in_specs += [full(w.shape) for w in weights]
    return pl.pallas_call(
        _edge_body,
        grid=(grid,),
        compiler_params=pltpu.CompilerParams(
            fuse_transposed_lhs_in_matmul=True),
        in_specs=in_specs,
        out_specs=[
            pl.BlockSpec((B, L), lambda i: (i, 0)),
            pl.BlockSpec((6, B), lambda i: (0, i)),
        ],
        out_shape=[
            jax.ShapeDtypeStruct((e, L), F32),
            jax.ShapeDtypeStruct((6, e), F32),
        ],
    )(stk, r0T, gs, gr, *weights)


def _scatter_stage(ridx, ft6, n):
    e = ridx.shape[0]
    ept = e // 16
    slc = -(-(3 * n) // 16)
    slc = -(-slc // 128) * 128
    acc_len = 16 * slc
    mesh = plsc.VectorSubcoreMesh(core_axis_name="c", subcore_axis_name="s")

    @functools.partial(
        pl.kernel, mesh=mesh,
        compiler_params=pltpu.CompilerParams(needs_layout_passes=False),
        out_type=jax.ShapeDtypeStruct((2 * acc_len,), F32),
        scratch_types=[
            pltpu.VMEM((GC,), jnp.int32),
            pltpu.VMEM((3 * GC,), F32),
            pltpu.VMEM((acc_len,), F32),
            pltpu.VMEM((slc,), F32),
            pltpu.VMEM_SHARED((16 * acc_len,), F32),
        ],
    )
    def k(ridx_hbm, ft_hbm, out_hbm, idxv, valv, acc, tmp, shared):
        c = lax.axis_index("c")
        t = lax.axis_index("s")
        z = jnp.zeros((16,), F32)

        def zb(i, carry):
            acc[pl.ds(i * 16, 16)] = z
            return carry

        lax.fori_loop(0, acc_len // 16, zb, 0)
        base = t * ept

        def body(i, carry):
            b = base + i * GC
            pltpu.sync_copy(ridx_hbm.at[pl.ds(b, GC)], idxv)
            for comp in range(3):
                pltpu.sync_copy(
                    ft_hbm.at[pl.ds((c * 3 + comp) * e + b, GC)],
                    valv.at[pl.ds(comp * GC, GC)])
            for sub in range(GC // 16):
                o = sub * 16
                r16 = idxv[pl.ds(o, 16)]
                a16 = r16 * 3
                for comp in range(3):
                    v16 = valv[pl.ds(comp * GC + o, 16)]
                    plsc.addupdate_scatter(acc, [a16 + comp], v16)
            return carry

        lax.fori_loop(0, ept // GC, body, 0)
        pltpu.sync_copy(acc, shared.at[pl.ds(t * acc_len, acc_len)])
        plsc.subcore_barrier()
        s0 = t * slc
        pltpu.sync_copy(shared.at[pl.ds(s0, slc)], tmp)

        def red(j, carry):
            pltpu.sync_copy(shared.at[pl.ds(j * acc_len + s0, slc)],
                            acc.at[pl.ds(0, slc)])

            def addl(q, carry2):
                tmp[pl.ds(q * 16, 16)] = (tmp[pl.ds(q * 16, 16)]
                                          + acc[pl.ds(q * 16, 16)])
                return carry2

            lax.fori_loop(0, slc // 16, addl, 0)
            return carry

        lax.fori_loop(1, 16, red, 0)
        pltpu.sync_copy(tmp, out_hbm.at[pl.ds(c * acc_len + s0, slc)])

    return k(ridx, ft6), acc_len


def _fin_body(accf_ref, acct_ref, ns_ref, dv_ref, dw_ref):
    minv = ns_ref[:, 1:2]
    iinv = ns_ref[:, 2:3]
    dvx = ns_ref[:, 3:4]
    dv_ref[...] = minv * accf_ref[...] + dvx
    dw_ref[...] = iinv * acct_ref[...]


def _fin_stage(accf, acct, nscal):
    n = nscal.shape[0]
    grid = n // FIN_BLK
    return pl.pallas_call(
        _fin_body,
        grid=(grid,),
        in_specs=[
            pl.BlockSpec((FIN_BLK, 3), lambda i: (i, 0)),
            pl.BlockSpec((FIN_BLK, 3), lambda i: (i, 0)),
            pl.BlockSpec((FIN_BLK, 4), lambda i: (i, 0)),
        ],
        out_specs=[
            pl.BlockSpec((FIN_BLK, 3), lambda i: (i, 0)),
            pl.BlockSpec((FIN_BLK, 3), lambda i: (i, 0)),
        ],
        out_shape=[
            jax.ShapeDtypeStruct((n, 3), F32),
            jax.ShapeDtypeStruct((n, 3), F32),
        ],
    )(accf, acct, nscal)


def _prep_node_weights(p):
    order = ("nw", "minv", "iinv", "dvext")
    w1cat = jnp.concatenate([p[k]["W1"] for k in order], axis=1)
    b1cat = jnp.concatenate([p[k]["b1"] for k in order])[None, :]
    w2cat = jnp.zeros((4 * L, 4), F32)
    for i, k in enumerate(order):
        w2cat = w2cat.at[i * L:(i + 1) * L, i:i + 1].set(p[k]["W2"])
    b2cat = jnp.concatenate([p[k]["b2"] for k in order])[None, :]
    return w1cat, b1cat, w2cat, b2cat


def _prep_edge_weights(p):
    ef, ed, it = p["edge_feat"], p["edge"], p["inter"]
    efW1 = jnp.concatenate([ef["W1"], jnp.zeros((16 - 9, L), F32)])
    eW1 = jnp.concatenate([ed["W1"], jnp.zeros((8 - 3, L), F32)])
    catW1 = jnp.concatenate([p["i1"]["W1"], p["i2"]["W1"], p["fsc"]["W1"]],
                            axis=1)
    catb1 = jnp.concatenate([p["i1"]["b1"], p["i2"]["b1"],
                             p["fsc"]["b1"]])[None, :]
    combW2 = jnp.zeros((3 * L, 8), F32)
    combW2 = combW2.at[0:L, 0:3].set(p["i1"]["W2"])
    combW2 = combW2.at[L:2 * L, 3:6].set(p["i2"]["W2"])
    combW2 = combW2.at[2 * L:3 * L, 6:7].set(p["fsc"]["W2"])
    combb2 = jnp.concatenate([p["i1"]["b2"], p["i2"]["b2"], p["fsc"]["b2"],
                              jnp.zeros((1,), F32)])[:, None]
    r1 = lambda x: x[None, :]
    return (efW1, r1(ef["b1"]), ef["W2"], r1(ef["b2"]), r1(ef["g"]),
            r1(ef["be"]),
            eW1, r1(ed["b1"]), ed["W2"], r1(ed["b2"]), r1(ed["g"]),
            r1(ed["be"]),
            it["W1"][0:L], it["W1"][L:2 * L], it["W1"][2 * L:3 * L],
            r1(it["b1"]), it["W2"],
            r1(it["b2"]), r1(it["g"]), r1(it["be"]),
            catW1, catb1, combW2, combb2)


def kernel(edge_index, senders_pos, receivers_pos, edge_dx_, edge_dt_,
           edge_attr, vector_a, vector_b, vector_c, senders_v_t_,
           senders_v_tm1_, senders_w_t_, receivers_v_t_, receivers_v_tm1_,
           receivers_w_t_, node_latent, params):
    n = node_latent.shape[0]
    ei = edge_index.astype(jnp.int32)
    senders, receivers = ei[0], ei[1]

    nw1, nb1, nw2, nb2 = _prep_node_weights(params)
    nl32 = node_latent.astype(F32)
    nscal = _node_stage(nl32, nw1, nb1, nw2, nb2)

    gs, gr, r0_flat = _gather_stage(
        nl32, senders, receivers, nscal[:, 0],
        senders_pos.T.reshape(-1), receivers_pos.T.reshape(-1))
    r0T = r0_flat.reshape(3, -1)

    stk = jnp.concatenate(
        [x.T for x in (senders_v_t_, senders_v_tm1_, senders_w_t_,
                       receivers_v_t_, receivers_v_tm1_, receivers_w_t_,
                       vector_a, vector_b, vector_c, edge_dx_, edge_dt_)]
        + [edge_attr.T, receivers_pos.T], axis=0)
    ew = _prep_edge_weights(params)
    il, ft6 = _edge_stage(stk, r0T, gs, gr, ew)

    acc2, acc_len = _scatter_stage(receivers, ft6.reshape(-1), n)
    accf = acc2[:3 * n].reshape(n, 3)
    acct = acc2[acc_len:acc_len + 3 * n].reshape(n, 3)

    dv, dw = _fin_stage(accf, acct, nscal)
    return (dv, dw, il)


# speedup vs baseline: 5.5618x; 1.3213x over previous
import functools

import jax
import jax.numpy as jnp
from jax import lax
from jax.experimental import pallas as pl
from jax.experimental.pallas import tpu as pltpu
from jax.experimental.pallas import tpu_sc as plsc

F32 = jnp.float32

L = 128
NODE_BLK = 2000
EDGE_BLK = 640
GC = 80
SC_WORKERS = 32
FIN_BLK = 2000
NSTK = 40


def _mm(a, w):
    return lax.dot_general(a, w, (((1,), (0,)), ((), ())),
                           preferred_element_type=F32)


def _mm_t(d, w):
    return lax.dot_general(d, w, (((0,), (0,)), ((), ())),
                           preferred_element_type=F32)


def _mm_h(w, a):
    return lax.dot_general(w, a, (((0,), (1,)), ((), ())),
                           preferred_element_type=F32)


def _ln(o, g, be):
    m = jnp.mean(o, axis=1, keepdims=True)
    c = o - m
    v = jnp.mean(c * c, axis=1, keepdims=True)
    return c * lax.rsqrt(v + 1e-5) * g + be


def _relu(x):
    return jnp.maximum(x, 0.0)


def _node_body(nl_ref, w1_ref, b1_ref, w2_ref, b2_ref, nscal_ref):
    nl = nl_ref[...]
    h = _relu(_mm(nl, w1_ref[...]) + b1_ref[...])
    nscal_ref[...] = _mm(h, w2_ref[...]) + b2_ref[...]


def _node_stage(node_latent, w1cat, b1cat, w2cat, b2cat):
    n = node_latent.shape[0]
    grid = n // NODE_BLK
    full = lambda shape: pl.BlockSpec(shape, lambda i: tuple(0 for _ in shape))
    return pl.pallas_call(
        _node_body,
        grid=(grid,),
        in_specs=[
            pl.BlockSpec((NODE_BLK, L), lambda i: (i, 0)),
            full((L, 4 * L)), full((1, 4 * L)), full((4 * L, 4)), full((1, 4)),
        ],
        out_specs=pl.BlockSpec((NODE_BLK, 4), lambda i: (i, 0)),
        out_shape=jax.ShapeDtypeStruct((n, 4), F32),
    )(node_latent, w1cat, b1cat, w2cat, b2cat)


def _gather_stage(tbl, sidx, ridx, nw_flat):
    e = sidx.shape[0]
    n = nw_flat.shape[0]
    epw = e // SC_WORKERS
    mesh = plsc.VectorSubcoreMesh(core_axis_name="c", subcore_axis_name="s")

    @functools.partial(
        pl.kernel, mesh=mesh,
        compiler_params=pltpu.CompilerParams(needs_layout_passes=False),
        out_type=[
            jax.ShapeDtypeStruct((e, L), F32),
            jax.ShapeDtypeStruct((e, L), F32),
            jax.ShapeDtypeStruct((2 * e,), F32),
        ],
        scratch_types=[
            pltpu.VMEM((n,), F32),
            pltpu.VMEM((epw,), jnp.int32),
            pltpu.VMEM((epw,), jnp.int32),
            pltpu.VMEM((GC, L), F32),
            pltpu.VMEM((GC, L), F32),
            pltpu.VMEM((GC, L), F32),
            pltpu.VMEM((GC, L), F32),
            pltpu.VMEM((epw,), F32),
            pltpu.VMEM((epw,), F32),
            pltpu.SemaphoreType.DMA,
            pltpu.SemaphoreType.DMA,
            pltpu.SemaphoreType.DMA,
            pltpu.SemaphoreType.DMA,
        ],
    )
    def k(tbl_hbm, s_hbm, r_hbm, nw_hbm, gs_hbm, gr_hbm, ww_hbm,
          nw_v, siv, riv, srows0, rrows0, srows1, rrows1, wsv, wrv,
          sg0, rg0, sg1, rg1):
        wid = lax.axis_index("s") * 2 + lax.axis_index("c")
        base = wid * epw
        pltpu.sync_copy(s_hbm.at[pl.ds(base, epw)], siv)
        pltpu.sync_copy(r_hbm.at[pl.ds(base, epw)], riv)
        pltpu.sync_copy(nw_hbm, nw_v)

        nchunk = epw // GC
        sbufs = (srows0, srows1)
        rbufs = (rrows0, rrows1)
        ssems = (sg0, sg1)
        rsems = (rg0, rg1)

        def issue(i, p):
            pltpu.async_copy(tbl_hbm.at[siv.at[pl.ds(i * GC, GC)]],
                             sbufs[p], ssems[p])
            pltpu.async_copy(tbl_hbm.at[riv.at[pl.ds(i * GC, GC)]],
                             rbufs[p], rsems[p])

        def drain_write(i, p):
            pltpu.make_async_copy(tbl_hbm.at[siv.at[pl.ds(0, GC)]],
                                  sbufs[p], ssems[p]).wait()
            pltpu.make_async_copy(tbl_hbm.at[riv.at[pl.ds(0, GC)]],
                                  rbufs[p], rsems[p]).wait()
            pltpu.sync_copy(sbufs[p], gs_hbm.at[pl.ds(base + i * GC, GC)])
            pltpu.sync_copy(rbufs[p], gr_hbm.at[pl.ds(base + i * GC, GC)])

        issue(0, 0)

        def wbody(q, carry):
            o = q * 16
            s16 = siv[pl.ds(o, 16)]
            r16 = riv[pl.ds(o, 16)]
            nws = plsc.load_gather(nw_v, [s16])
            nwr = plsc.load_gather(nw_v, [r16])
            inv = 1.0 / (nws + nwr)
            wsv[pl.ds(o, 16)] = nws * inv
            wrv[pl.ds(o, 16)] = nwr * inv
            return carry

        lax.fori_loop(0, epw // 16, wbody, 0)
        pltpu.sync_copy(wsv, ww_hbm.at[pl.ds(base, epw)])
        pltpu.sync_copy(wrv, ww_hbm.at[pl.ds(e + base, epw)])

        def body(i2, carry):
            i = 1 + 2 * i2
            issue(i, 1)
            drain_write(i - 1, 0)
            issue(i + 1, 0)
            drain_write(i, 1)
            return carry

        lax.fori_loop(0, (nchunk - 1) // 2, body, 0)
        drain_write(nchunk - 1, 0)

    return k(tbl, sidx, ridx, nw_flat)


def _edge_body(stk_ref, ww_ref, gs_ref, gr_ref,
               efW1_ref, efb1_ref, efW2_ref, efb2_ref, efg_ref, efbe_ref,
               eW1_ref, eb1_ref, eW2_ref, eb2_ref, eg_ref, ebe_ref,
               iW1a_ref, iW1b_ref, iW1c_ref, ib1_ref, iW2_ref, ib2_ref,
               ig_ref, ibe_ref, catW1_ref, catb1_ref, combW2_ref, combb2_ref,
               il_ref, ft_ref):
    B = gs_ref.shape[0]

    def row(k):
        return stk_ref[k, :]

    def rowm(ref, k):
        return ref[k, :].reshape(1, B)

    zrow = jnp.zeros((1, B), F32)

    def dstack(rows9):
        rows = [d.reshape(1, B) for d in rows9]
        return jnp.concatenate(rows + [zrow] * (16 - len(rows)), axis=0)

    uvecs = [[row(9 * g + 3 * i + c) for c in range(3)] for g in range(2)
             for i in range(3)]
    vvecs = [[row(18 + 3 * j + c) for c in range(3)] for j in range(3)]

    def dot3(u, v):
        return u[0] * v[0] + u[1] * v[1] + u[2] * v[2]

    ds = dstack([dot3(uvecs[i], vvecs[j])
                 for i in range(3) for j in range(3)])
    dr = dstack([-dot3(uvecs[3 + i], vvecs[j])
                 for i in range(3) for j in range(3)])

    acc_s = _mm_t(ds, efW1_ref[...]) + efb1_ref[...]
    acc_r = _mm_t(dr, efW1_ref[...]) + efb1_ref[...]

    efW2, efb2, efg, efbe = efW2_ref[...], efb2_ref[...], efg_ref[...], efbe_ref[...]
    s_lat = _ln(_mm(_relu(acc_s), efW2) + efb2, efg, efbe)
    r_lat = _ln(_mm(_relu(acc_r), efW2) + efb2, efg, efbe)

    dxc = [row(27 + c) for c in range(3)]
    dtc = [row(30 + c) for c in range(3)]
    dxn = jnp.sqrt(dot3(dxc, dxc))
    dtn = jnp.sqrt(dot3(dtc, dtc))
    de = jnp.concatenate([dxn.reshape(1, B), dtn.reshape(1, B),
                          rowm(stk_ref, 33)] + [zrow] * 5, axis=0)
    acc_e = _mm_t(de, eW1_ref[...]) + eb1_ref[...]
    e_lat = _ln(_mm(_relu(acc_e), eW2_ref[...]) + eb2_ref[...],
                eg_ref[...], ebe_ref[...])

    pre = (_mm(s_lat + r_lat, iW1a_ref[...])
           + _mm(gs_ref[...] + gr_ref[...], iW1b_ref[...])
           + _mm(e_lat, iW1c_ref[...]) + ib1_ref[...])
    il = _ln(_mm(_relu(pre), iW2_ref[...]) + ib2_ref[...],
             ig_ref[...], ibe_ref[...])
    il_ref[...] = il

    hcat = _relu(_mm(il, catW1_ref[...]) + catb1_ref[...])
    cfT = _mm_h(combW2_ref[...], hcat) + combb2_ref[...]

    cf = [cfT[j, :] for j in range(7)]
    va = [row(18 + c) for c in range(3)]
    vb = [row(21 + c) for c in range(3)]
    vc = [row(24 + c) for c in range(3)]
    f = [cf[0] * va[c] + cf[1] * vb[c] + cf[2] * vc[c] for c in range(3)]
    a = [cf[3] * va[c] + cf[4] * vb[c] + cf[5] * vc[c] for c in range(3)]
    lam = cf[6]

    rp = [row(34 + c) for c in range(3)]
    sp = [row(37 + c) for c in range(3)]
    ws = ww_ref[0, :]
    wr = ww_ref[1, :]
    d = [rp[c] - (ws * sp[c] + wr * rp[c]) for c in range(3)]
    cr = [d[1] * f[2] - d[2] * f[1],
          d[2] * f[0] - d[0] * f[2],
          d[0] * f[1] - d[1] * f[0]]

    for c in range(3):
        ft_ref[c, :] = f[c]
        ft_ref[3 + c, :] = a[c] - cr[c] * lam


def _edge_stage(stk, ww, gs, gr, weights):
    e = gs.shape[0]
    B = EDGE_BLK
    grid = e // B
    full = lambda shape: pl.BlockSpec(shape, lambda i: tuple(0 for _ in shape))
    in_specs = [
        pl.BlockSpec((NSTK, B), lambda i: (0, i)),
        pl.BlockSpec((2, B), lambda i: (0, i)),
        pl.BlockSpec((B, L), lambda i: (i, 0)),
        pl.BlockSpec((B, L), lambda i: (i, 0)),
    ]
    in_specs += [full(w.shape) for w in weights]
    return pl.pallas_call(
        _edge_body,
        grid=(grid,),
        compiler_params=pltpu.CompilerParams(
            fuse_transposed_lhs_in_matmul=True),
        in_specs=in_specs,
        out_specs=[
            pl.BlockSpec((B, L), lambda i: (i, 0)),
            pl.BlockSpec((6, B), lambda i: (0, i)),
        ],
        out_shape=[
            jax.ShapeDtypeStruct((e, L), F32),
            jax.ShapeDtypeStruct((6, e), F32),
        ],
    )(stk, ww, gs, gr, *weights)


def _scatter_stage(ridx, ft6, n):
    e = ridx.shape[0]
    ept = e // 16
    slc = -(-(3 * n) // 16)
    slc = -(-slc // 128) * 128
    acc_len = 16 * slc
    mesh = plsc.VectorSubcoreMesh(core_axis_name="c", subcore_axis_name="s")

    @functools.partial(
        pl.kernel, mesh=mesh,
        compiler_params=pltpu.CompilerParams(needs_layout_passes=False),
        out_type=jax.ShapeDtypeStruct((2 * acc_len,), F32),
        scratch_types=[
            pltpu.VMEM((ept,), jnp.int32),
            pltpu.VMEM((3 * (ept // 2),), F32),
            pltpu.VMEM((acc_len,), F32),
            pltpu.VMEM((slc,), F32),
            pltpu.VMEM_SHARED((16 * acc_len,), F32),
        ],
    )
    def k(ridx_hbm, ft_hbm, out_hbm, idxv, valv, acc, tmp, shared):
        c = lax.axis_index("c")
        t = lax.axis_index("s")
        z = jnp.zeros((16,), F32)

        def zb(i, carry):
            acc[pl.ds(i * 16, 16)] = z
            return carry

        lax.fori_loop(0, acc_len // 16, zb, 0)
        base = t * ept
        hb = ept // 2
        pltpu.sync_copy(ridx_hbm.at[pl.ds(base, ept)], idxv)
        for half in range(2):
            for comp in range(3):
                pltpu.sync_copy(
                    ft_hbm.at[pl.ds((c * 3 + comp) * e + base + half * hb, hb)],
                    valv.at[pl.ds(comp * hb, hb)])

            def body(sub, carry):
                o = sub * 16
                r16 = idxv[pl.ds(half * hb + o, 16)]
                a16 = r16 * 3
                for comp in range(3):
                    v16 = valv[pl.ds(comp * hb + o, 16)]
                    plsc.addupdate_scatter(acc, [a16 + comp], v16)
                return carry

            lax.fori_loop(0, hb // 16, body, 0)
        pltpu.sync_copy(acc, shared.at[pl.ds(t * acc_len, acc_len)])
        plsc.subcore_barrier()
        s0 = t * slc
        pltpu.sync_copy(shared.at[pl.ds(s0, slc)], tmp)

        def red(j, carry):
            pltpu.sync_copy(shared.at[pl.ds(j * acc_len + s0, slc)],
                            acc.at[pl.ds(0, slc)])

            def addl(q, carry2):
                tmp[pl.ds(q * 16, 16)] = (tmp[pl.ds(q * 16, 16)]
                                          + acc[pl.ds(q * 16, 16)])
                return carry2

            lax.fori_loop(0, slc // 16, addl, 0)
            return carry

        lax.fori_loop(1, 16, red, 0)
        pltpu.sync_copy(tmp, out_hbm.at[pl.ds(c * acc_len + s0, slc)])

    return k(ridx, ft6), acc_len


def _fin_body(accf_ref, acct_ref, ns_ref, dv_ref, dw_ref):
    minv = ns_ref[:, 1:2]
    iinv = ns_ref[:, 2:3]
    dvx = ns_ref[:, 3:4]
    dv_ref[...] = minv * accf_ref[...] + dvx
    dw_ref[...] = iinv * acct_ref[...]


def _fin_stage(accf, acct, nscal):
    n = nscal.shape[0]
    grid = n // FIN_BLK
    return pl.pallas_call(
        _fin_body,
        grid=(grid,),
        in_specs=[
            pl.BlockSpec((FIN_BLK, 3), lambda i: (i, 0)),
            pl.BlockSpec((FIN_BLK, 3), lambda i: (i, 0)),
            pl.BlockSpec((FIN_BLK, 4), lambda i: (i, 0)),
        ],
        out_specs=[
            pl.BlockSpec((FIN_BLK, 3), lambda i: (i, 0)),
            pl.BlockSpec((FIN_BLK, 3), lambda i: (i, 0)),
        ],
        out_shape=[
            jax.ShapeDtypeStruct((n, 3), F32),
            jax.ShapeDtypeStruct((n, 3), F32),
        ],
    )(accf, acct, nscal)


def _prep_node_weights(p):
    order = ("nw", "minv", "iinv", "dvext")
    w1cat = jnp.concatenate([p[k]["W1"] for k in order], axis=1)
    b1cat = jnp.concatenate([p[k]["b1"] for k in order])[None, :]
    w2cat = jnp.zeros((4 * L, 4), F32)
    for i, k in enumerate(order):
        w2cat = w2cat.at[i * L:(i + 1) * L, i:i + 1].set(p[k]["W2"])
    b2cat = jnp.concatenate([p[k]["b2"] for k in order])[None, :]
    return w1cat, b1cat, w2cat, b2cat


def _prep_edge_weights(p):
    ef, ed, it = p["edge_feat"], p["edge"], p["inter"]
    efW1 = jnp.concatenate([ef["W1"], jnp.zeros((16 - 9, L), F32)])
    eW1 = jnp.concatenate([ed["W1"], jnp.zeros((8 - 3, L), F32)])
    catW1 = jnp.concatenate([p["i1"]["W1"], p["i2"]["W1"], p["fsc"]["W1"]],
                            axis=1)
    catb1 = jnp.concatenate([p["i1"]["b1"], p["i2"]["b1"],
                             p["fsc"]["b1"]])[None, :]
    combW2 = jnp.zeros((3 * L, 8), F32)
    combW2 = combW2.at[0:L, 0:3].set(p["i1"]["W2"])
    combW2 = combW2.at[L:2 * L, 3:6].set(p["i2"]["W2"])
    combW2 = combW2.at[2 * L:3 * L, 6:7].set(p["fsc"]["W2"])
    combb2 = jnp.concatenate([p["i1"]["b2"], p["i2"]["b2"], p["fsc"]["b2"],
                              jnp.zeros((1,), F32)])[:, None]
    r1 = lambda x: x[None, :]
    return (efW1, r1(ef["b1"]), ef["W2"], r1(ef["b2"]), r1(ef["g"]),
            r1(ef["be"]),
            eW1, r1(ed["b1"]), ed["W2"], r1(ed["b2"]), r1(ed["g"]),
            r1(ed["be"]),
            it["W1"][0:L], it["W1"][L:2 * L], it["W1"][2 * L:3 * L],
            r1(it["b1"]), it["W2"],
            r1(it["b2"]), r1(it["g"]), r1(it["be"]),
            catW1, catb1, combW2, combb2)


def kernel(edge_index, senders_pos, receivers_pos, edge_dx_, edge_dt_,
           edge_attr, vector_a, vector_b, vector_c, senders_v_t_,
           senders_v_tm1_, senders_w_t_, receivers_v_t_, receivers_v_tm1_,
           receivers_w_t_, node_latent, params):
    n = node_latent.shape[0]
    ei = edge_index.astype(jnp.int32)
    senders, receivers = ei[0], ei[1]

    nw1, nb1, nw2, nb2 = _prep_node_weights(params)
    nl32 = node_latent.astype(F32)
    nscal = _node_stage(nl32, nw1, nb1, nw2, nb2)

    gs, gr, ww_flat = _gather_stage(nl32, senders, receivers, nscal[:, 0])
    ww = ww_flat.reshape(2, -1)

    stk = jnp.concatenate(
        [x.T for x in (senders_v_t_, senders_v_tm1_, senders_w_t_,
                       receivers_v_t_, receivers_v_tm1_, receivers_w_t_,
                       vector_a, vector_b, vector_c, edge_dx_, edge_dt_)]
        + [edge_attr.T, receivers_pos.T, senders_pos.T], axis=0)
    ew = _prep_edge_weights(params)
    il, ft6 = _edge_stage(stk, ww, gs, gr, ew)

    acc2, acc_len = _scatter_stage(receivers, ft6.reshape(-1), n)
    accf = acc2[:3 * n].reshape(n, 3)
    acct = acc2[acc_len:acc_len + 3 * n].reshape(n, 3)

    dv, dw = _fin_stage(accf, acct, nscal)
    return (dv, dw, il)
